# baseline (speedup 1.0000x reference)
import functools

import jax
import jax.numpy as jnp
from jax import lax
from jax.experimental import pallas as pl
from jax.experimental.pallas import tpu as pltpu
from jax.experimental.pallas import tpu_sc as plsc

N = 10000
E = 320000
D = 128
L = 3

NC = 2
NS = 16
NW = NC * NS
K = 100
NB = E // K
NB_W = NB // NW
NP = 10240
NROW_T = NP // NS


DH = D // 2


def _sc_segment_sum(tableA, tableB, src3d, dst3d, zeros):
    mesh = plsc.VectorSubcoreMesh(core_axis_name="c", subcore_axis_name="s")

    @functools.partial(
        pl.kernel,
        out_type=jax.ShapeDtypeStruct((2 * NC, NP, DH), jnp.float32),
        mesh=mesh,
        scratch_types=[
            pltpu.VMEM((NB_W, K), jnp.int32),
            pltpu.VMEM((NB_W, K), jnp.int32),
            pltpu.VMEM((4 * K, DH), jnp.float32),
            pltpu.VMEM_SHARED((NP, DH), jnp.float32),
            pltpu.SemaphoreType.DMA((4,)),
        ],
        compiler_params=pltpu.CompilerParams(use_tc_tiling_on_sc=False),
    )
    def k(tableA_hbm, tableB_hbm, src_hbm, dst_hbm, zeros_hbm, out_hbm,
          src_v, dst_v, rows_v, acc_sh, sems):
        c = lax.axis_index("c")
        s = lax.axis_index("s")
        wid = c * NS + s

        pltpu.sync_copy(src_hbm.at[wid], src_v)
        pltpu.sync_copy(dst_hbm.at[wid], dst_v)

        for p, table_hbm in enumerate((tableA_hbm, tableB_hbm)):
            pltpu.sync_copy(zeros_hbm.at[pl.ds(s * NROW_T, NROW_T)],
                            acc_sh.at[pl.ds(s * NROW_T, NROW_T)])
            plsc.subcore_barrier()

            def body(i, carry2, table_hbm=table_hbm):
                b = lax.rem(i, 4)

                @pl.when(i >= 4)
                def _():
                    pltpu.make_async_copy(table_hbm.at[pl.ds(0, K)],
                                          rows_v.at[pl.ds(0, K)],
                                          sems.at[b]).wait()
                    pltpu.sync_copy(rows_v.at[pl.ds(b * K, K)],
                                    acc_sh.at[dst_v.at[i - 4]], add=True)

                @pl.when(i < NB_W)
                def _():
                    pltpu.async_copy(table_hbm.at[src_v.at[i]],
                                     rows_v.at[pl.ds(b * K, K)], sems.at[b])
                return carry2

            lax.fori_loop(0, NB_W + 4, body, 0, unroll=False)
            plsc.subcore_barrier()

            pltpu.sync_copy(acc_sh.at[pl.ds(s * NROW_T, NROW_T)],
                            out_hbm.at[c * 2 + p].at[pl.ds(s * NROW_T,
                                                           NROW_T)])

    return k(tableA, tableB, src3d, dst3d, zeros)


_BLK = 1000


def _tc_embed(x, W, b):
    def body(x_ref, w_ref, b_ref, o_ref):
        o_ref[...] = (
            jnp.dot(x_ref[...], w_ref[...], preferred_element_type=jnp.float32)
            + b_ref[...]
        )

    return pl.pallas_call(
        body,
        grid=(N // _BLK,),
        in_specs=[
            pl.BlockSpec((_BLK, D), lambda i: (i, 0)),
            pl.BlockSpec((D, D), lambda i: (0, 0)),
            pl.BlockSpec((1, D), lambda i: (0, 0)),
        ],
        out_specs=pl.BlockSpec((_BLK, D), lambda i: (i, 0)),
        out_shape=jax.ShapeDtypeStruct((N, D), jnp.float32),
    )(x, W, b.reshape(1, D))


def _tc_self(inv, W_self, b_conv, W_feat, b_feat):
    def body(inv_ref, ws_ref, bc_ref, wf_ref, bf_ref, o_ref):
        h = jnp.dot(inv_ref[...], ws_ref[...], preferred_element_type=jnp.float32)
        h = h + bc_ref[...]
        o_ref[...] = (
            jnp.dot(h, wf_ref[...], preferred_element_type=jnp.float32)
            + bf_ref[...]
        )

    mat = pl.BlockSpec((_BLK, D), lambda i: (i, 0))
    wspec = pl.BlockSpec((D, D), lambda i: (0, 0))
    bspec = pl.BlockSpec((1, D), lambda i: (0, 0))
    return pl.pallas_call(
        body,
        grid=(N // _BLK,),
        in_specs=[mat, wspec, bspec, wspec, bspec],
        out_specs=mat,
        out_shape=jax.ShapeDtypeStruct((N, D), jnp.float32),
    )(inv, W_self, b_conv.reshape(1, D), W_feat, b_feat.reshape(1, D))


def _tc_combine(hs2, acc4, W_neigh, W_feat):
    def body(hs_ref, a_ref, wn_ref, wf_ref, o_ref):
        wnf = jnp.dot(wn_ref[...], wf_ref[...], preferred_element_type=jnp.float32)
        agg_h0 = a_ref[0] + a_ref[2]
        agg_h1 = a_ref[1] + a_ref[3]
        o = hs_ref[...]
        o = o + jnp.dot(agg_h0, wnf[0:DH, :], preferred_element_type=jnp.float32)
        o = o + jnp.dot(agg_h1, wnf[DH:D, :], preferred_element_type=jnp.float32)
        o_ref[...] = jnp.maximum(o, 0.0)

    mat = pl.BlockSpec((_BLK, D), lambda i: (i, 0))
    aspec = pl.BlockSpec((2 * NC, _BLK, DH), lambda i: (0, i, 0))
    wspec = pl.BlockSpec((D, D), lambda i: (0, 0))
    return pl.pallas_call(
        body,
        grid=(N // _BLK,),
        in_specs=[mat, aspec, wspec, wspec],
        out_specs=mat,
        out_shape=jax.ShapeDtypeStruct((N, D), jnp.float32),
    )(hs2, acc4, W_neigh, W_feat)


def kernel(x, pos, edge_index, W_emb, b_emb, W_self, W_neigh, b_conv,
           W_feat, b_feat):
    src3d = edge_index[0].astype(jnp.int32).reshape(NW, NB_W, K)
    dst3d = edge_index[1].astype(jnp.int32).reshape(NW, NB_W, K)
    zeros = jnp.zeros((NP, DH), jnp.float32)

    inv = _tc_embed(x, W_emb, b_emb)
    for l in range(L):
        acc4 = _sc_segment_sum(inv[:, :DH], inv[:, DH:], src3d, dst3d, zeros)
        hs2 = _tc_self(inv, W_self[l], b_conv[l], W_feat[l], b_feat[l])
        inv = _tc_combine(hs2, acc4, W_neigh[l], W_feat[l])
    return (inv, pos, edge_index)


# speedup vs baseline: 1.0848x; 1.0848x over previous
import functools

import jax
import jax.numpy as jnp
from jax import lax
from jax.experimental import pallas as pl
from jax.experimental.pallas import tpu as pltpu
from jax.experimental.pallas import tpu_sc as plsc

N = 10000
E = 320000
D = 128
L = 3

NC = 2
NS = 16
NW = NC * NS
K = 100
NB = E // K
NB_W = NB // NW
NP = 10240
NROW_T = NP // NS


DH = D // 2


def _sc_segment_sum(table2, srcs, dst3d, zeros):
    mesh = plsc.VectorSubcoreMesh(core_axis_name="c", subcore_axis_name="s")

    @functools.partial(
        pl.kernel,
        out_type=jax.ShapeDtypeStruct((2 * NC, NP, DH), jnp.float32),
        mesh=mesh,
        scratch_types=[
            pltpu.VMEM((2 * NB_W, K), jnp.int32),
            pltpu.VMEM((NB_W, K), jnp.int32),
            pltpu.VMEM((8 * K, DH), jnp.float32),
            pltpu.VMEM_SHARED((NP, DH), jnp.float32),
            pltpu.SemaphoreType.DMA((8,)),
        ],
        compiler_params=pltpu.CompilerParams(use_tc_tiling_on_sc=False),
    )
    def k(table_hbm, src_hbm, dst_hbm, zeros_hbm, out_hbm,
          src_v, dst_v, rows_v, acc_sh, sems):
        c = lax.axis_index("c")
        s = lax.axis_index("s")
        wid = c * NS + s

        pltpu.sync_copy(src_hbm.at[wid], src_v)
        pltpu.sync_copy(dst_hbm.at[wid], dst_v)

        def one_pass(p, carry):
            pltpu.sync_copy(zeros_hbm.at[pl.ds(s * NROW_T, NROW_T)],
                            acc_sh.at[pl.ds(s * NROW_T, NROW_T)])
            plsc.subcore_barrier()

            def body(i, carry2):
                b = lax.rem(i, 8)

                @pl.when(i >= 8)
                def _():
                    pltpu.make_async_copy(table_hbm.at[pl.ds(0, K)],
                                          rows_v.at[pl.ds(0, K)],
                                          sems.at[b]).wait()
                    pltpu.sync_copy(rows_v.at[pl.ds(b * K, K)],
                                    acc_sh.at[dst_v.at[i - 8]], add=True)

                @pl.when(i < NB_W)
                def _():
                    pltpu.async_copy(table_hbm.at[src_v.at[p * NB_W + i]],
                                     rows_v.at[pl.ds(b * K, K)], sems.at[b])
                return carry2

            lax.fori_loop(0, NB_W + 8, body, 0, unroll=False)
            plsc.subcore_barrier()

            pltpu.sync_copy(acc_sh.at[pl.ds(s * NROW_T, NROW_T)],
                            out_hbm.at[c * 2 + p].at[pl.ds(s * NROW_T,
                                                           NROW_T)])
            return carry

        lax.fori_loop(0, 2, one_pass, 0, unroll=False)

    return k(table2, srcs, dst3d, zeros)


_BLK = 1000


def _tc_embed(x, W, b):
    def body(x_ref, w_ref, b_ref, o_ref):
        o_ref[...] = (
            jnp.dot(x_ref[...], w_ref[...], preferred_element_type=jnp.float32)
            + b_ref[...]
        )

    return pl.pallas_call(
        body,
        grid=(N // _BLK,),
        in_specs=[
            pl.BlockSpec((_BLK, D), lambda i: (i, 0)),
            pl.BlockSpec((D, D), lambda i: (0, 0)),
            pl.BlockSpec((1, D), lambda i: (0, 0)),
        ],
        out_specs=pl.BlockSpec((_BLK, D), lambda i: (i, 0)),
        out_shape=jax.ShapeDtypeStruct((N, D), jnp.float32),
    )(x, W, b.reshape(1, D))


def _tc_self(inv, W_self, b_conv, W_feat, b_feat):
    def body(inv_ref, ws_ref, bc_ref, wf_ref, bf_ref, o_ref):
        h = jnp.dot(inv_ref[...], ws_ref[...], preferred_element_type=jnp.float32)
        h = h + bc_ref[...]
        o_ref[...] = (
            jnp.dot(h, wf_ref[...], preferred_element_type=jnp.float32)
            + bf_ref[...]
        )

    mat = pl.BlockSpec((_BLK, D), lambda i: (i, 0))
    wspec = pl.BlockSpec((D, D), lambda i: (0, 0))
    bspec = pl.BlockSpec((1, D), lambda i: (0, 0))
    return pl.pallas_call(
        body,
        grid=(N // _BLK,),
        in_specs=[mat, wspec, bspec, wspec, bspec],
        out_specs=mat,
        out_shape=jax.ShapeDtypeStruct((N, D), jnp.float32),
    )(inv, W_self, b_conv.reshape(1, D), W_feat, b_feat.reshape(1, D))


def _tc_combine(hs2, acc4, W_neigh, W_feat):
    def body(hs_ref, a_ref, wn_ref, wf_ref, o_ref):
        wnf = jnp.dot(wn_ref[...], wf_ref[...], preferred_element_type=jnp.float32)
        agg_h0 = a_ref[0] + a_ref[2]
        agg_h1 = a_ref[1] + a_ref[3]
        o = hs_ref[...]
        o = o + jnp.dot(agg_h0, wnf[0:DH, :], preferred_element_type=jnp.float32)
        o = o + jnp.dot(agg_h1, wnf[DH:D, :], preferred_element_type=jnp.float32)
        o_ref[...] = jnp.maximum(o, 0.0)

    mat = pl.BlockSpec((_BLK, D), lambda i: (i, 0))
    aspec = pl.BlockSpec((2 * NC, _BLK, DH), lambda i: (0, i, 0))
    wspec = pl.BlockSpec((D, D), lambda i: (0, 0))
    return pl.pallas_call(
        body,
        grid=(N // _BLK,),
        in_specs=[mat, aspec, wspec, wspec],
        out_specs=mat,
        out_shape=jax.ShapeDtypeStruct((N, D), jnp.float32),
    )(hs2, acc4, W_neigh, W_feat)


def kernel(x, pos, edge_index, W_emb, b_emb, W_self, W_neigh, b_conv,
           W_feat, b_feat):
    src = edge_index[0].astype(jnp.int32)
    sE = (2 * src).reshape(NW, NB_W, K)
    sO = (2 * src + 1).reshape(NW, NB_W, K)
    srcs = jnp.concatenate([sE, sO], axis=1)
    dst3d = edge_index[1].astype(jnp.int32).reshape(NW, NB_W, K)
    zeros = jnp.zeros((NP, DH), jnp.float32)

    inv = _tc_embed(x, W_emb, b_emb)
    for l in range(L):
        acc4 = _sc_segment_sum(inv.reshape(2 * N, DH), srcs, dst3d, zeros)
        hs2 = _tc_self(inv, W_self[l], b_conv[l], W_feat[l], b_feat[l])
        inv = _tc_combine(hs2, acc4, W_neigh[l], W_feat[l])
    return (inv, pos, edge_index)
